# baseline (speedup 1.0000x reference)
import jax
import jax.numpy as jnp
from jax.experimental import pallas as pl
from jax.experimental.pallas import tpu as pltpu

H = 16
NC = 6
NL = 4
N = 512
P = 8
TI = 32
NJJ = N // P


def _silu(v):
    return v * jax.nn.sigmoid(v)


def _pack(t16):
    return jnp.concatenate([t16[NJJ * g:NJJ * (g + 1)] for g in range(P)],
                           axis=1)


def _egnn_body(xref, wref, at_ref, eiW, eib, eoW, eob, fcW, fcb, t8, ff,
               wrt, whc, whrt, cbias, e2m, e2b, c1m, c1b, c2r,
               n1a, n1b, n1bb, n2m, n2bb, out_ref):
    f32 = jnp.float32
    x = xref[0]
    wq = wref[0]
    h = jnp.dot(at_ref[:], eiW[:], preferred_element_type=f32) + eib[0]
    T8 = t8[:]
    F = ff[:]
    F1 = F[:, 0:1]
    for l in range(NL):
        xn = jnp.sum(x * x, axis=1, keepdims=True)
        wl = wrt[l]
        xks = [_pack(jnp.tile(x[:, k:k + 1], (1, H))) for k in range(3)]
        xnp_ = _pack(jnp.tile(xn, (1, H)))
        hc = jnp.dot(h, whc[l], preferred_element_type=f32)
        hc128 = _pack(hc)
        row012 = jnp.stack([-2.0 * wl[0] * xk for xk in xks], axis=1)
        row3 = jnp.broadcast_to(wl.reshape(1, 1, P * H), (NJJ, 1, P * H))
        row4 = (xnp_ * wl[0] + hc128 + cbias[l, 0])[:, None, :]
        rows_h = jnp.broadcast_to(whrt[l][None], (NJJ, H, P * H))
        rhs = jnp.concatenate([row012, row3, row4, rows_h], axis=1)
        ah_parts = []
        xu_parts = []
        for it in range(N // TI):
            i0 = it * TI
            xt = x[i0:i0 + TI]
            lhs = jnp.concatenate(
                [xt, xn[i0:i0 + TI], jnp.ones((TI, 1), f32),
                 h[i0:i0 + TI]], axis=1)
            lhsb = jnp.broadcast_to(lhs[None], (NJJ, TI, 21))
            pre3 = jax.lax.dot_general(
                lhsb, rhs, (((2,), (1,)), ((0,), (0,))),
                preferred_element_type=f32)
            m1 = _silu(pre3)
            m = _silu(jax.lax.dot_general(
                m1, e2m[l], (((2,), (0,)), ((), ())),
                preferred_element_type=f32) + e2b[l, 0])
            tt = _silu(jax.lax.dot_general(
                m, c1m[l], (((2,), (0,)), ((), ())),
                preferred_element_type=f32) + c1b[l, 0])
            cm8 = jax.lax.dot_general(
                tt, c2r[l], (((2,), (0,)), ((), ())),
                preferred_element_type=f32)
            cm128 = jax.lax.dot_general(
                cm8, T8, (((2,), (0,)), ((), ())),
                preferred_element_type=f32)
            wsl = (wq[:, :, i0:i0 + TI].astype(jnp.int32) > 0).astype(f32)
            wpk = jax.lax.dot_general(
                wsl, T8, (((1,), (0,)), ((), ())),
                preferred_element_type=f32)
            ah128 = jnp.sum(m * wpk, axis=0)
            ah_parts.append(jnp.dot(ah128, F, preferred_element_type=f32))
            cmw128 = cm128 * wpk
            vks = [jnp.dot(jnp.sum(cmw128 * xk[:, None, :], axis=0), F1,
                           preferred_element_type=f32) for xk in xks]
            v = jnp.concatenate(vks, axis=1)
            s1 = jnp.dot(jnp.sum(cmw128, axis=0), F1,
                         preferred_element_type=f32)
            cnt_t = jnp.dot(jnp.sum(wpk, axis=0), F1,
                            preferred_element_type=f32)
            xnum = xt * s1 - v
            xu_parts.append(xnum / jnp.maximum(cnt_t, 1.0))
        ah = jnp.concatenate(ah_parts, axis=0)
        xupd = jnp.concatenate(xu_parts, axis=0)
        o = _silu(jnp.dot(h, n1a[l], preferred_element_type=f32)
                  + jnp.dot(ah, n1b[l], preferred_element_type=f32)
                  + n1bb[l, 0])
        h = h + jnp.dot(o, n2m[l], preferred_element_type=f32) + n2bb[l, 0]
        x = x + xupd
    h = jnp.dot(h, eoW[:], preferred_element_type=f32) + eob[0]
    prot = jnp.sum(h, axis=0, keepdims=True) * (1.0 / N)
    lg = jnp.dot(prot, fcW[:], preferred_element_type=f32) + fcb[0]
    lg = lg - jnp.max(lg, axis=1, keepdims=True)
    ex = jnp.exp(lg)
    out_ref[...] = (ex / jnp.sum(ex, axis=1, keepdims=True)).reshape(1, 1, NC)


def kernel(data, params):
    B = data.shape[0]
    p = params
    f32 = jnp.float32
    xsrc = data[:, :, 0:3]
    wsrc = data[:, :, 3:].reshape(B, N, P, NJJ).transpose(0, 3, 2, 1)
    eye8 = jnp.eye(P, dtype=f32)
    T8 = jnp.kron(eye8, jnp.ones((1, H), f32))
    F = jnp.kron(jnp.ones((P, 1), f32), jnp.eye(H, dtype=f32))
    lays = p['layers']
    wrt = jnp.stack([jnp.tile(lp['e1_W'][2 * H].reshape(1, H), (1, P))
                     for lp in lays])
    whc = jnp.stack([lp['e1_W'][H:2 * H] for lp in lays])
    whrt = jnp.stack([jnp.tile(lp['e1_W'][0:H], (1, P))
                      for lp in lays])
    cbias = jnp.stack([jnp.tile((lp['e1_W'][2 * H + 1]
                                 + lp['e1_b']).reshape(1, H), (1, P))
                       for lp in lays])
    e2m = jnp.stack([jnp.kron(eye8, lp['e2_W']) for lp in lays])
    e2b = jnp.stack([jnp.tile(lp['e2_b'].reshape(1, H), (1, P))
                     for lp in lays])
    c1m = jnp.stack([jnp.kron(eye8, lp['c1_W']) for lp in lays])
    c1b = jnp.stack([jnp.tile(lp['c1_b'].reshape(1, H), (1, P))
                     for lp in lays])
    c2r = jnp.stack([jnp.kron(eye8, lp['c2_W']) for lp in lays])
    n1a = jnp.stack([lp['n1_W'][0:H] for lp in lays])
    n1b = jnp.stack([lp['n1_W'][H:2 * H] for lp in lays])
    n1bb = jnp.stack([lp['n1_b'].reshape(1, H) for lp in lays])
    n2m = jnp.stack([lp['n2_W'] for lp in lays])
    n2bb = jnp.stack([lp['n2_b'].reshape(1, H) for lp in lays])

    ops = [p['atom_table'], p['emb_in_W'], p['emb_in_b'].reshape(1, H),
           p['emb_out_W'], p['emb_out_b'].reshape(1, H),
           p['fc_W'], p['fc_b'].reshape(1, NC), T8, F,
           wrt, whc, whrt, cbias, e2m, e2b, c1m, c1b, c2r,
           n1a, n1b, n1bb, n2m, n2bb]

    def _full(a):
        nd = a.ndim
        return pl.BlockSpec(a.shape, lambda b, _n=nd: (0,) * _n)

    out = pl.pallas_call(
        _egnn_body,
        grid=(B,),
        in_specs=[pl.BlockSpec((1, N, 3), lambda b: (b, 0, 0)),
                  pl.BlockSpec((1, NJJ, P, N), lambda b: (b, 0, 0, 0))]
                 + [_full(a) for a in ops],
        out_specs=pl.BlockSpec((1, 1, NC), lambda b: (b, 0, 0)),
        out_shape=jax.ShapeDtypeStruct((B, 1, NC), jnp.float32),
        compiler_params=pltpu.CompilerParams(
            dimension_semantics=("parallel",),
            vmem_limit_bytes=60 * 2 ** 20),
    )(xsrc, wsrc, *ops)
    return out.reshape(B, NC)


# speedup vs baseline: 1.0328x; 1.0328x over previous
import jax
import jax.numpy as jnp
from jax.experimental import pallas as pl
from jax.experimental.pallas import tpu as pltpu

H = 16
NC = 6
NL = 4
N = 512
P = 8
TI = 32
NJJ = N // P


def _silu(v):
    return v * jax.nn.sigmoid(v)


def _pack(t16):
    return jnp.concatenate([t16[NJJ * g:NJJ * (g + 1)] for g in range(P)],
                           axis=1)


def _egnn_body(xref, wref, at_ref, eiW, eib, eoW, eob, fcW, fcb, t8, ff,
               wrt, whc, whrt, cbias, e2m, e2b, c1m, c1b, c2r,
               n1a, n1b, n1bb, n2m, n2bb, out_ref):
    f32 = jnp.float32
    x = xref[0]
    wq = wref[0]
    h = jnp.dot(at_ref[:], eiW[:], preferred_element_type=f32) + eib[0]
    T8 = t8[:]
    F = ff[:]
    F1 = F[:, 0:1]
    for l in range(NL):
        xn = jnp.sum(x * x, axis=1, keepdims=True)
        wl = wrt[l]
        xks = [_pack(jnp.tile(x[:, k:k + 1], (1, H))) for k in range(3)]
        xnp_ = _pack(jnp.tile(xn, (1, H)))
        hc = jnp.dot(h, whc[l], preferred_element_type=f32)
        hc128 = _pack(hc)
        row012 = jnp.stack([-2.0 * wl[0] * xk for xk in xks], axis=1)
        row3 = jnp.broadcast_to(wl.reshape(1, 1, P * H), (NJJ, 1, P * H))
        row4 = (xnp_ * wl[0] + hc128 + cbias[l, 0])[:, None, :]
        rows_h = jnp.broadcast_to(whrt[l][None], (NJJ, H, P * H))
        rhs = jnp.concatenate([row012, row3, row4, rows_h], axis=1)
        ah_parts = []
        xu_parts = []
        for it in range(N // TI):
            i0 = it * TI
            xt = x[i0:i0 + TI]
            lhs = jnp.concatenate(
                [xt, xn[i0:i0 + TI], jnp.ones((TI, 1), f32),
                 h[i0:i0 + TI]], axis=1)
            lhsb = jnp.broadcast_to(lhs[None], (NJJ, TI, 21))
            pre3 = jax.lax.dot_general(
                lhsb, rhs, (((2,), (1,)), ((0,), (0,))),
                preferred_element_type=f32)
            m1 = _silu(pre3).reshape(NJJ * TI, P * H)
            m2 = _silu(jnp.dot(m1, e2m[l], preferred_element_type=f32)
                       + e2b[l, 0])
            tt = _silu(jnp.dot(m2, c1m[l], preferred_element_type=f32)
                       + c1b[l, 0])
            m = m2.reshape(NJJ, TI, P * H)
            cm128 = jnp.dot(tt, c2r[l],
                            preferred_element_type=f32).reshape(NJJ, TI,
                                                                P * H)
            wsl = (wq[:, :, i0:i0 + TI].astype(jnp.int32) > 0).astype(f32)
            wpk = jax.lax.dot_general(
                wsl, T8, (((1,), (0,)), ((), ())),
                preferred_element_type=f32)
            ah128 = jnp.sum(m * wpk, axis=0)
            ah_parts.append(jnp.dot(ah128, F, preferred_element_type=f32))
            cmw128 = cm128 * wpk
            vks = [jnp.dot(jnp.sum(cmw128 * xk[:, None, :], axis=0), F1,
                           preferred_element_type=f32) for xk in xks]
            v = jnp.concatenate(vks, axis=1)
            s1 = jnp.dot(jnp.sum(cmw128, axis=0), F1,
                         preferred_element_type=f32)
            cnt_t = jnp.dot(jnp.sum(wpk, axis=0), F1,
                            preferred_element_type=f32)
            xnum = xt * s1 - v
            xu_parts.append(xnum / jnp.maximum(cnt_t, 1.0))
        ah = jnp.concatenate(ah_parts, axis=0)
        xupd = jnp.concatenate(xu_parts, axis=0)
        o = _silu(jnp.dot(h, n1a[l], preferred_element_type=f32)
                  + jnp.dot(ah, n1b[l], preferred_element_type=f32)
                  + n1bb[l, 0])
        h = h + jnp.dot(o, n2m[l], preferred_element_type=f32) + n2bb[l, 0]
        x = x + xupd
    h = jnp.dot(h, eoW[:], preferred_element_type=f32) + eob[0]
    prot = jnp.sum(h, axis=0, keepdims=True) * (1.0 / N)
    lg = jnp.dot(prot, fcW[:], preferred_element_type=f32) + fcb[0]
    lg = lg - jnp.max(lg, axis=1, keepdims=True)
    ex = jnp.exp(lg)
    out_ref[...] = (ex / jnp.sum(ex, axis=1, keepdims=True)).reshape(1, 1, NC)


def kernel(data, params):
    B = data.shape[0]
    p = params
    f32 = jnp.float32
    xsrc = data[:, :, 0:3]
    wsrc = data[:, :, 3:].reshape(B, N, P, NJJ).transpose(0, 3, 2, 1)
    eye8 = jnp.eye(P, dtype=f32)
    T8 = jnp.kron(eye8, jnp.ones((1, H), f32))
    F = jnp.kron(jnp.ones((P, 1), f32), jnp.eye(H, dtype=f32))
    lays = p['layers']
    wrt = jnp.stack([jnp.tile(lp['e1_W'][2 * H].reshape(1, H), (1, P))
                     for lp in lays])
    whc = jnp.stack([lp['e1_W'][H:2 * H] for lp in lays])
    whrt = jnp.stack([jnp.tile(lp['e1_W'][0:H], (1, P))
                      for lp in lays])
    cbias = jnp.stack([jnp.tile((lp['e1_W'][2 * H + 1]
                                 + lp['e1_b']).reshape(1, H), (1, P))
                       for lp in lays])
    e2m = jnp.stack([jnp.kron(eye8, lp['e2_W']) for lp in lays])
    e2b = jnp.stack([jnp.tile(lp['e2_b'].reshape(1, H), (1, P))
                     for lp in lays])
    c1m = jnp.stack([jnp.kron(eye8, lp['c1_W']) for lp in lays])
    c1b = jnp.stack([jnp.tile(lp['c1_b'].reshape(1, H), (1, P))
                     for lp in lays])
    c2r = jnp.stack([jnp.kron(eye8, lp['c2_W'] @ jnp.ones((1, H), f32))
                     for lp in lays])
    n1a = jnp.stack([lp['n1_W'][0:H] for lp in lays])
    n1b = jnp.stack([lp['n1_W'][H:2 * H] for lp in lays])
    n1bb = jnp.stack([lp['n1_b'].reshape(1, H) for lp in lays])
    n2m = jnp.stack([lp['n2_W'] for lp in lays])
    n2bb = jnp.stack([lp['n2_b'].reshape(1, H) for lp in lays])

    ops = [p['atom_table'], p['emb_in_W'], p['emb_in_b'].reshape(1, H),
           p['emb_out_W'], p['emb_out_b'].reshape(1, H),
           p['fc_W'], p['fc_b'].reshape(1, NC), T8, F,
           wrt, whc, whrt, cbias, e2m, e2b, c1m, c1b, c2r,
           n1a, n1b, n1bb, n2m, n2bb]

    def _full(a):
        nd = a.ndim
        return pl.BlockSpec(a.shape, lambda b, _n=nd: (0,) * _n)

    out = pl.pallas_call(
        _egnn_body,
        grid=(B,),
        in_specs=[pl.BlockSpec((1, N, 3), lambda b: (b, 0, 0)),
                  pl.BlockSpec((1, NJJ, P, N), lambda b: (b, 0, 0, 0))]
                 + [_full(a) for a in ops],
        out_specs=pl.BlockSpec((1, 1, NC), lambda b: (b, 0, 0)),
        out_shape=jax.ShapeDtypeStruct((B, 1, NC), jnp.float32),
        compiler_params=pltpu.CompilerParams(
            dimension_semantics=("parallel",),
            vmem_limit_bytes=60 * 2 ** 20),
    )(xsrc, wsrc, *ops)
    return out.reshape(B, NC)


# speedup vs baseline: 1.0344x; 1.0015x over previous
import jax
import jax.numpy as jnp
from jax.experimental import pallas as pl
from jax.experimental.pallas import tpu as pltpu

H = 16
NC = 6
NL = 4
N = 512
P = 8
TI = 32
NJJ = N // P


def _silu(v):
    return v * jax.nn.sigmoid(v)


def _pack(t16):
    return jnp.concatenate([t16[NJJ * g:NJJ * (g + 1)] for g in range(P)],
                           axis=1)


def _egnn_body(xref, wref, at_ref, eiW, eib, eoW, eob, fcW, fcb, t8, ff,
               wrt, whc, whrt, cbias, e2m, e2b, c1m, c1b, c2r,
               n1a, n1b, n1bb, n2m, n2bb, out_ref):
    f32 = jnp.float32
    x = xref[0]
    wq = wref[0]
    h = jnp.dot(at_ref[:], eiW[:], preferred_element_type=f32) + eib[0]
    T8 = t8[:]
    F = ff[:]
    F1 = F[:, 0:1]
    for l in range(NL):
        xn = jnp.sum(x * x, axis=1, keepdims=True)
        wl = wrt[l]
        xks = [_pack(jnp.tile(x[:, k:k + 1], (1, H))) for k in range(3)]
        xnp_ = _pack(jnp.tile(xn, (1, H)))
        hc = jnp.dot(h, whc[l], preferred_element_type=f32)
        hc128 = _pack(hc)
        row012 = jnp.stack([-2.0 * wl[0] * xk for xk in xks], axis=1)
        row3 = jnp.broadcast_to(wl.reshape(1, 1, P * H), (NJJ, 1, P * H))
        row4 = (xnp_ * wl[0] + hc128 + cbias[l, 0])[:, None, :]
        rows_h = jnp.broadcast_to(whrt[l][None], (NJJ, H, P * H))
        rhs = jnp.concatenate([row012, row3, row4, rows_h], axis=1)
        ah_parts = []
        xu_parts = []
        for it in range(N // TI):
            i0 = it * TI
            xt = x[i0:i0 + TI]
            lhs = jnp.concatenate(
                [xt, xn[i0:i0 + TI], jnp.ones((TI, 1), f32),
                 h[i0:i0 + TI]], axis=1)
            lhsb = jnp.broadcast_to(lhs[None], (NJJ, TI, 21))
            pre3 = jax.lax.dot_general(
                lhsb, rhs, (((2,), (1,)), ((0,), (0,))),
                preferred_element_type=f32)
            m1 = _silu(pre3).reshape(NJJ * TI, P * H)
            m2 = _silu(jnp.dot(m1.astype(jnp.bfloat16), e2m[l],
                               preferred_element_type=f32) + e2b[l, 0])
            tt = _silu(jnp.dot(m2.astype(jnp.bfloat16), c1m[l],
                               preferred_element_type=f32) + c1b[l, 0])
            m = m2.reshape(NJJ, TI, P * H)
            cm128 = jnp.dot(tt.astype(jnp.bfloat16), c2r[l],
                            preferred_element_type=f32).reshape(NJJ, TI,
                                                                P * H)
            wsl = (wq[:, :, i0:i0 + TI].astype(jnp.int32) > 0).astype(f32)
            wpk = jax.lax.dot_general(
                wsl, T8, (((1,), (0,)), ((), ())),
                preferred_element_type=f32)
            ah128 = jnp.sum(m * wpk, axis=0)
            ah_parts.append(jnp.dot(ah128, F, preferred_element_type=f32))
            cmw128 = cm128 * wpk
            vks = [jnp.dot(jnp.sum(cmw128 * xk[:, None, :], axis=0), F1,
                           preferred_element_type=f32) for xk in xks]
            v = jnp.concatenate(vks, axis=1)
            s1 = jnp.dot(jnp.sum(cmw128, axis=0), F1,
                         preferred_element_type=f32)
            cnt_t = jnp.dot(jnp.sum(wpk, axis=0), F1,
                            preferred_element_type=f32)
            xnum = xt * s1 - v
            xu_parts.append(xnum / jnp.maximum(cnt_t, 1.0))
        ah = jnp.concatenate(ah_parts, axis=0)
        xupd = jnp.concatenate(xu_parts, axis=0)
        o = _silu(jnp.dot(h, n1a[l], preferred_element_type=f32)
                  + jnp.dot(ah, n1b[l], preferred_element_type=f32)
                  + n1bb[l, 0])
        h = h + jnp.dot(o, n2m[l], preferred_element_type=f32) + n2bb[l, 0]
        x = x + xupd
    h = jnp.dot(h, eoW[:], preferred_element_type=f32) + eob[0]
    prot = jnp.sum(h, axis=0, keepdims=True) * (1.0 / N)
    lg = jnp.dot(prot, fcW[:], preferred_element_type=f32) + fcb[0]
    lg = lg - jnp.max(lg, axis=1, keepdims=True)
    ex = jnp.exp(lg)
    out_ref[...] = (ex / jnp.sum(ex, axis=1, keepdims=True)).reshape(1, 1, NC)


def kernel(data, params):
    B = data.shape[0]
    p = params
    f32 = jnp.float32
    xsrc = data[:, :, 0:3]
    wsrc = data[:, :, 3:].reshape(B, N, P, NJJ).transpose(0, 3, 2, 1)
    eye8 = jnp.eye(P, dtype=f32)
    T8 = jnp.kron(eye8, jnp.ones((1, H), f32))
    F = jnp.kron(jnp.ones((P, 1), f32), jnp.eye(H, dtype=f32))
    lays = p['layers']
    wrt = jnp.stack([jnp.tile(lp['e1_W'][2 * H].reshape(1, H), (1, P))
                     for lp in lays])
    whc = jnp.stack([lp['e1_W'][H:2 * H] for lp in lays])
    whrt = jnp.stack([jnp.tile(lp['e1_W'][0:H], (1, P))
                      for lp in lays])
    cbias = jnp.stack([jnp.tile((lp['e1_W'][2 * H + 1]
                                 + lp['e1_b']).reshape(1, H), (1, P))
                       for lp in lays])
    e2m = jnp.stack([jnp.kron(eye8, lp['e2_W'])
                     for lp in lays]).astype(jnp.bfloat16)
    e2b = jnp.stack([jnp.tile(lp['e2_b'].reshape(1, H), (1, P))
                     for lp in lays])
    c1m = jnp.stack([jnp.kron(eye8, lp['c1_W'])
                     for lp in lays]).astype(jnp.bfloat16)
    c1b = jnp.stack([jnp.tile(lp['c1_b'].reshape(1, H), (1, P))
                     for lp in lays])
    c2r = jnp.stack([jnp.kron(eye8, lp['c2_W'] @ jnp.ones((1, H), f32))
                     for lp in lays]).astype(jnp.bfloat16)
    n1a = jnp.stack([lp['n1_W'][0:H] for lp in lays])
    n1b = jnp.stack([lp['n1_W'][H:2 * H] for lp in lays])
    n1bb = jnp.stack([lp['n1_b'].reshape(1, H) for lp in lays])
    n2m = jnp.stack([lp['n2_W'] for lp in lays])
    n2bb = jnp.stack([lp['n2_b'].reshape(1, H) for lp in lays])

    ops = [p['atom_table'], p['emb_in_W'], p['emb_in_b'].reshape(1, H),
           p['emb_out_W'], p['emb_out_b'].reshape(1, H),
           p['fc_W'], p['fc_b'].reshape(1, NC), T8, F,
           wrt, whc, whrt, cbias, e2m, e2b, c1m, c1b, c2r,
           n1a, n1b, n1bb, n2m, n2bb]

    def _full(a):
        nd = a.ndim
        return pl.BlockSpec(a.shape, lambda b, _n=nd: (0,) * _n)

    out = pl.pallas_call(
        _egnn_body,
        grid=(B,),
        in_specs=[pl.BlockSpec((1, N, 3), lambda b: (b, 0, 0)),
                  pl.BlockSpec((1, NJJ, P, N), lambda b: (b, 0, 0, 0))]
                 + [_full(a) for a in ops],
        out_specs=pl.BlockSpec((1, 1, NC), lambda b: (b, 0, 0)),
        out_shape=jax.ShapeDtypeStruct((B, 1, NC), jnp.float32),
        compiler_params=pltpu.CompilerParams(
            dimension_semantics=("parallel",),
            vmem_limit_bytes=60 * 2 ** 20),
    )(xsrc, wsrc, *ops)
    return out.reshape(B, NC)


# speedup vs baseline: 1.0427x; 1.0080x over previous
import jax
import jax.numpy as jnp
from jax.experimental import pallas as pl
from jax.experimental.pallas import tpu as pltpu

H = 16
NC = 6
NL = 4
N = 512
P = 8
TI = 32
NJJ = N // P


def _silu(v):
    return v * jax.nn.sigmoid(v)


def _pack(t16):
    return jnp.concatenate([t16[NJJ * g:NJJ * (g + 1)] for g in range(P)],
                           axis=1)


def _egnn_body(xref, wref, at_ref, eiW, eib, eoW, eob, fcW, fcb, t8, ff,
               wrt, whc, whrt, cbias, e2m, e2b, c1m, c1b, c2r,
               n1a, n1b, n1bb, n2m, n2bb, out_ref):
    f32 = jnp.float32
    x = xref[0]
    wq = wref[0]
    h = jnp.dot(at_ref[:], eiW[:], preferred_element_type=f32) + eib[0]
    T8 = t8[:]
    F = ff[:]
    F1 = F[:, 0:1]
    for l in range(NL):
        xn = jnp.sum(x * x, axis=1, keepdims=True)
        wl = wrt[l]
        xks = [_pack(jnp.tile(x[:, k:k + 1], (1, H))) for k in range(3)]
        xnp_ = _pack(jnp.tile(xn, (1, H)))
        hc = jnp.dot(h, whc[l], preferred_element_type=f32)
        hc128 = _pack(hc)
        rhs012 = jnp.stack([-2.0 * wl[0] * xk for xk in xks], axis=1)
        acol = xnp_ * wl[0] + hc128 + cbias[l, 0]
        brow = (jnp.dot(h, whrt[l], preferred_element_type=f32)
                + xn * wl)
        ah_parts = []
        xu_parts = []
        for it in range(N // TI):
            i0 = it * TI
            xt = x[i0:i0 + TI]
            lhsb = jnp.broadcast_to(xt[None], (NJJ, TI, 3))
            pre3 = (jax.lax.dot_general(
                lhsb, rhs012, (((2,), (1,)), ((0,), (0,))),
                preferred_element_type=f32)
                + brow[i0:i0 + TI][None, :, :] + acol[:, None, :])
            m1 = _silu(pre3).reshape(NJJ * TI, P * H)
            m2 = _silu(jnp.dot(m1, e2m[l], preferred_element_type=f32)
                       + e2b[l, 0])
            tt = _silu(jnp.dot(m2, c1m[l], preferred_element_type=f32)
                       + c1b[l, 0])
            m = m2.reshape(NJJ, TI, P * H)
            cm128 = jnp.dot(tt, c2r[l],
                            preferred_element_type=f32).reshape(NJJ, TI,
                                                                P * H)
            wsl = (wq[:, :, i0:i0 + TI].astype(jnp.int32) > 0).astype(f32)
            wpk = jax.lax.dot_general(
                wsl, T8, (((1,), (0,)), ((), ())),
                preferred_element_type=f32)
            ah128 = jnp.sum(m * wpk, axis=0)
            ah_parts.append(jnp.dot(ah128, F, preferred_element_type=f32))
            cmw128 = cm128 * wpk
            vks = [jnp.dot(jnp.sum(cmw128 * xk[:, None, :], axis=0), F1,
                           preferred_element_type=f32) for xk in xks]
            v = jnp.concatenate(vks, axis=1)
            s1 = jnp.dot(jnp.sum(cmw128, axis=0), F1,
                         preferred_element_type=f32)
            cnt_t = jnp.dot(jnp.sum(wpk, axis=0), F1,
                            preferred_element_type=f32)
            xnum = xt * s1 - v
            xu_parts.append(xnum / jnp.maximum(cnt_t, 1.0))
        ah = jnp.concatenate(ah_parts, axis=0)
        xupd = jnp.concatenate(xu_parts, axis=0)
        o = _silu(jnp.dot(h, n1a[l], preferred_element_type=f32)
                  + jnp.dot(ah, n1b[l], preferred_element_type=f32)
                  + n1bb[l, 0])
        h = h + jnp.dot(o, n2m[l], preferred_element_type=f32) + n2bb[l, 0]
        x = x + xupd
    h = jnp.dot(h, eoW[:], preferred_element_type=f32) + eob[0]
    prot = jnp.sum(h, axis=0, keepdims=True) * (1.0 / N)
    lg = jnp.dot(prot, fcW[:], preferred_element_type=f32) + fcb[0]
    lg = lg - jnp.max(lg, axis=1, keepdims=True)
    ex = jnp.exp(lg)
    out_ref[...] = (ex / jnp.sum(ex, axis=1, keepdims=True)).reshape(1, 1, NC)


def kernel(data, params):
    B = data.shape[0]
    p = params
    f32 = jnp.float32
    xsrc = data[:, :, 0:3]
    wsrc = data[:, :, 3:].reshape(B, N, P, NJJ).transpose(0, 3, 2, 1)
    eye8 = jnp.eye(P, dtype=f32)
    T8 = jnp.kron(eye8, jnp.ones((1, H), f32))
    F = jnp.kron(jnp.ones((P, 1), f32), jnp.eye(H, dtype=f32))
    lays = p['layers']
    wrt = jnp.stack([jnp.tile(lp['e1_W'][2 * H].reshape(1, H), (1, P))
                     for lp in lays])
    whc = jnp.stack([lp['e1_W'][H:2 * H] for lp in lays])
    whrt = jnp.stack([jnp.tile(lp['e1_W'][0:H], (1, P))
                      for lp in lays])
    cbias = jnp.stack([jnp.tile((lp['e1_W'][2 * H + 1]
                                 + lp['e1_b']).reshape(1, H), (1, P))
                       for lp in lays])
    e2m = jnp.stack([jnp.kron(eye8, lp['e2_W']) for lp in lays])
    e2b = jnp.stack([jnp.tile(lp['e2_b'].reshape(1, H), (1, P))
                     for lp in lays])
    c1m = jnp.stack([jnp.kron(eye8, lp['c1_W']) for lp in lays])
    c1b = jnp.stack([jnp.tile(lp['c1_b'].reshape(1, H), (1, P))
                     for lp in lays])
    c2r = jnp.stack([jnp.kron(eye8, lp['c2_W'] @ jnp.ones((1, H), f32))
                     for lp in lays])
    n1a = jnp.stack([lp['n1_W'][0:H] for lp in lays])
    n1b = jnp.stack([lp['n1_W'][H:2 * H] for lp in lays])
    n1bb = jnp.stack([lp['n1_b'].reshape(1, H) for lp in lays])
    n2m = jnp.stack([lp['n2_W'] for lp in lays])
    n2bb = jnp.stack([lp['n2_b'].reshape(1, H) for lp in lays])

    ops = [p['atom_table'], p['emb_in_W'], p['emb_in_b'].reshape(1, H),
           p['emb_out_W'], p['emb_out_b'].reshape(1, H),
           p['fc_W'], p['fc_b'].reshape(1, NC), T8, F,
           wrt, whc, whrt, cbias, e2m, e2b, c1m, c1b, c2r,
           n1a, n1b, n1bb, n2m, n2bb]

    def _full(a):
        nd = a.ndim
        return pl.BlockSpec(a.shape, lambda b, _n=nd: (0,) * _n)

    out = pl.pallas_call(
        _egnn_body,
        grid=(B,),
        in_specs=[pl.BlockSpec((1, N, 3), lambda b: (b, 0, 0)),
                  pl.BlockSpec((1, NJJ, P, N), lambda b: (b, 0, 0, 0))]
                 + [_full(a) for a in ops],
        out_specs=pl.BlockSpec((1, 1, NC), lambda b: (b, 0, 0)),
        out_shape=jax.ShapeDtypeStruct((B, 1, NC), jnp.float32),
        compiler_params=pltpu.CompilerParams(
            dimension_semantics=("parallel",),
            vmem_limit_bytes=60 * 2 ** 20),
    )(xsrc, wsrc, *ops)
    return out.reshape(B, NC)


# speedup vs baseline: 1.1142x; 1.0685x over previous
import jax
import jax.numpy as jnp
from jax.experimental import pallas as pl
from jax.experimental.pallas import tpu as pltpu

H = 16
NC = 6
NL = 4
N = 512
P = 8
TI = 32
NJJ = N // P


def _silu(v):
    return v * jax.nn.sigmoid(v)


def _pack(t16):
    return jnp.concatenate([t16[NJJ * g:NJJ * (g + 1)] for g in range(P)],
                           axis=1)


def _egnn_body(xref, wref, at_ref, eiW, eib, eoW, eob, fcW, fcb, t8, ff,
               wrt, whc, whrt, cbias, e2m, e2b, c1m, c1b, c2r,
               n1a, n1b, n1bb, n2m, n2bb, out_ref):
    f32 = jnp.float32
    x = xref[0]
    wq = wref[0]
    h = jnp.dot(at_ref[:], eiW[:], preferred_element_type=f32) + eib[0]
    T8 = t8[:]
    F = ff[:]
    F1 = F[:, 0:1]
    F4 = F[:, 0:4]
    cnt_tiles = []
    for l in range(NL):
        xn = jnp.sum(x * x, axis=1, keepdims=True)
        wl = wrt[l]
        xks = [_pack(jnp.tile(x[:, k:k + 1], (1, H))) for k in range(3)]
        xnp_ = _pack(jnp.tile(xn, (1, H)))
        hc = jnp.dot(h, whc[l], preferred_element_type=f32)
        hc128 = _pack(hc)
        rhs012 = jnp.stack([-2.0 * wl[0] * xk for xk in xks], axis=1)
        xc3 = _pack(jnp.concatenate(
            [x, jnp.ones((N, 1), f32), jnp.zeros((N, H - 4), f32)],
            axis=1))
        acol = xnp_ * wl[0] + hc128 + cbias[l, 0]
        brow = (jnp.dot(h, whrt[l], preferred_element_type=f32)
                + xn * wl)
        ah_parts = []
        xu_parts = []
        for it in range(N // TI):
            i0 = it * TI
            xt = x[i0:i0 + TI]
            lhsb = jnp.broadcast_to(xt[None], (NJJ, TI, 3))
            pre3 = (jax.lax.dot_general(
                lhsb, rhs012, (((2,), (1,)), ((0,), (0,))),
                preferred_element_type=f32)
                + brow[i0:i0 + TI][None, :, :] + acol[:, None, :])
            m1 = _silu(pre3).reshape(NJJ * TI, P * H)
            m2 = _silu(jnp.dot(m1, e2m[l], preferred_element_type=f32)
                       + e2b[l, 0])
            tt = _silu(jnp.dot(m2, c1m[l], preferred_element_type=f32)
                       + c1b[l, 0])
            m = m2.reshape(NJJ, TI, P * H)
            cm128 = jnp.dot(tt, c2r[l],
                            preferred_element_type=f32).reshape(NJJ, TI,
                                                                P * H)
            wsl = (wq[:, :, i0:i0 + TI].astype(jnp.int32) > 0).astype(f32)
            wpk = jax.lax.dot_general(
                wsl, T8, (((1,), (0,)), ((), ())),
                preferred_element_type=f32)
            ah128 = jnp.sum(m * wpk, axis=0)
            ah_parts.append(jnp.dot(ah128, F, preferred_element_type=f32))
            cmw128 = cm128 * wpk
            vs4 = jnp.dot(jnp.sum(cmw128 * xc3[:, None, :], axis=0), F4,
                          preferred_element_type=f32)
            v = vs4[:, 0:3]
            s1 = vs4[:, 3:4]
            if l == 0:
                cnt_t = jnp.maximum(
                    jnp.dot(jnp.sum(wpk, axis=0), F1,
                            preferred_element_type=f32), 1.0)
                cnt_tiles.append(cnt_t)
            else:
                cnt_t = cnt_tiles[it]
            xnum = xt * s1 - v
            xu_parts.append(xnum / cnt_t)
        ah = jnp.concatenate(ah_parts, axis=0)
        xupd = jnp.concatenate(xu_parts, axis=0)
        o = _silu(jnp.dot(h, n1a[l], preferred_element_type=f32)
                  + jnp.dot(ah, n1b[l], preferred_element_type=f32)
                  + n1bb[l, 0])
        h = h + jnp.dot(o, n2m[l], preferred_element_type=f32) + n2bb[l, 0]
        x = x + xupd
    h = jnp.dot(h, eoW[:], preferred_element_type=f32) + eob[0]
    prot = jnp.sum(h, axis=0, keepdims=True) * (1.0 / N)
    lg = jnp.dot(prot, fcW[:], preferred_element_type=f32) + fcb[0]
    lg = lg - jnp.max(lg, axis=1, keepdims=True)
    ex = jnp.exp(lg)
    out_ref[...] = (ex / jnp.sum(ex, axis=1, keepdims=True)).reshape(1, 1, NC)


def kernel(data, params):
    B = data.shape[0]
    p = params
    f32 = jnp.float32
    xsrc = data[:, :, 0:3]
    wsrc = data[:, :, 3:].reshape(B, N, P, NJJ).transpose(0, 3, 2, 1)
    eye8 = jnp.eye(P, dtype=f32)
    T8 = jnp.kron(eye8, jnp.ones((1, H), f32))
    F = jnp.kron(jnp.ones((P, 1), f32), jnp.eye(H, dtype=f32))
    lays = p['layers']
    wrt = jnp.stack([jnp.tile(lp['e1_W'][2 * H].reshape(1, H), (1, P))
                     for lp in lays])
    whc = jnp.stack([lp['e1_W'][H:2 * H] for lp in lays])
    whrt = jnp.stack([jnp.tile(lp['e1_W'][0:H], (1, P))
                      for lp in lays])
    cbias = jnp.stack([jnp.tile((lp['e1_W'][2 * H + 1]
                                 + lp['e1_b']).reshape(1, H), (1, P))
                       for lp in lays])
    e2m = jnp.stack([jnp.kron(eye8, lp['e2_W']) for lp in lays])
    e2b = jnp.stack([jnp.tile(lp['e2_b'].reshape(1, H), (1, P))
                     for lp in lays])
    c1m = jnp.stack([jnp.kron(eye8, lp['c1_W']) for lp in lays])
    c1b = jnp.stack([jnp.tile(lp['c1_b'].reshape(1, H), (1, P))
                     for lp in lays])
    c2r = jnp.stack([jnp.kron(eye8, lp['c2_W'] @ jnp.ones((1, H), f32))
                     for lp in lays])
    n1a = jnp.stack([lp['n1_W'][0:H] for lp in lays])
    n1b = jnp.stack([lp['n1_W'][H:2 * H] for lp in lays])
    n1bb = jnp.stack([lp['n1_b'].reshape(1, H) for lp in lays])
    n2m = jnp.stack([lp['n2_W'] for lp in lays])
    n2bb = jnp.stack([lp['n2_b'].reshape(1, H) for lp in lays])

    ops = [p['atom_table'], p['emb_in_W'], p['emb_in_b'].reshape(1, H),
           p['emb_out_W'], p['emb_out_b'].reshape(1, H),
           p['fc_W'], p['fc_b'].reshape(1, NC), T8, F,
           wrt, whc, whrt, cbias, e2m, e2b, c1m, c1b, c2r,
           n1a, n1b, n1bb, n2m, n2bb]

    def _full(a):
        nd = a.ndim
        return pl.BlockSpec(a.shape, lambda b, _n=nd: (0,) * _n)

    out = pl.pallas_call(
        _egnn_body,
        grid=(B,),
        in_specs=[pl.BlockSpec((1, N, 3), lambda b: (b, 0, 0)),
                  pl.BlockSpec((1, NJJ, P, N), lambda b: (b, 0, 0, 0))]
                 + [_full(a) for a in ops],
        out_specs=pl.BlockSpec((1, 1, NC), lambda b: (b, 0, 0)),
        out_shape=jax.ShapeDtypeStruct((B, 1, NC), jnp.float32),
        compiler_params=pltpu.CompilerParams(
            dimension_semantics=("parallel",),
            vmem_limit_bytes=60 * 2 ** 20),
    )(xsrc, wsrc, *ops)
    return out.reshape(B, NC)
